# Initial kernel scaffold; baseline (speedup 1.0000x reference)
import jax
import jax.numpy as jnp
from jax.experimental import pallas as pl


def kernel(x, edge_index, batch, edge_attr, node_W, node_b, edge_W, edge_b, mlp_W1, mlp_b1, mlp_W2, mlp_b2, eps, bn_g, bn_b, cls_W1, cls_b1, cls_W2, cls_b2):
    raise NotImplementedError("write your pallas kernel here")


# speedup vs baseline: 1.7595x; 1.7595x over previous
import functools

import jax
import jax.numpy as jnp
from jax import lax
from jax.experimental import pallas as pl
from jax.experimental.pallas import tpu as pltpu
import jax.experimental.pallas.tpu_sc as plsc

N = 10000
E = 320000
H = 256
HH = 128
L = 5
NG = 64
NCLS = 10

NSC = 2
NT = 16
B = 64
BH = 128
NBLK = 320
EPT = NBLK * B
NROW = EPT // BH
E_PAD = NT * EPT
AROWS = 10240


def _edge_body(h_hbm, e_hbm, src_hbm, dst_hbm, out_hbm,
               srcb, dstb, hbuf, ebuf, aggr_sh, isem, hsem, esem):
    c = lax.axis_index("c")
    s = lax.axis_index("s")
    coff = c * N
    ebase = c * E_PAD + s * EPT

    z0 = hbuf.at[0]

    def _zero(r, carry):
        for k in range(HH // 16):
            z0[r, pl.ds(k * 16, 16)] = jnp.zeros((16,), jnp.float32)
        return carry
    lax.fori_loop(0, B, _zero, 0)
    zbase = s * 640
    for k in range(640 // B):
        pltpu.sync_copy(z0, aggr_sh.at[pl.ds(zbase + k * B, B)])
    plsc.subcore_barrier()

    def _idx_src_refs(i):
        r = lax.div(i, 2)
        off = lax.rem(i, 2) * B
        return (src_hbm.at[s, r, pl.ds(off, B)], dst_hbm.at[s, r, pl.ds(off, B)])

    def _issue_idx(i):
        sl = lax.rem(i, 4)
        sh, dh = _idx_src_refs(i)
        pltpu.async_copy(sh, srcb.at[sl], isem)
        pltpu.async_copy(dh, dstb.at[sl], isem)

    def _wait_idx(i):
        sl = lax.rem(i, 4)
        sh, dh = _idx_src_refs(i)
        pltpu.make_async_copy(sh, srcb.at[sl], isem).wait()
        pltpu.make_async_copy(dh, dstb.at[sl], isem).wait()
        for k in range(B // 16):
            ds = pl.ds(k * 16, 16)
            srcb[sl, ds] = srcb[sl, ds] + coff

    def _issue_data(i):
        sl = lax.rem(i, 2)
        pltpu.async_copy(h_hbm.at[srcb.at[lax.rem(i, 4)]], hbuf.at[sl], hsem)
        pltpu.async_copy(e_hbm.at[pl.ds(ebase + i * B, B)], ebuf.at[sl], esem)

    def _wait_data(i):
        sl = lax.rem(i, 2)
        pltpu.make_async_copy(h_hbm.at[srcb.at[lax.rem(i, 4)]],
                              hbuf.at[sl], hsem).wait()
        pltpu.make_async_copy(e_hbm.at[pl.ds(ebase + i * B, B)],
                              ebuf.at[sl], esem).wait()

    _issue_idx(0)
    _issue_idx(1)
    _wait_idx(0)
    _issue_data(0)

    def _blk(i, carry):
        @pl.when(i + 2 < NBLK)
        def _():
            _issue_idx(i + 2)

        @pl.when(i + 1 < NBLK)
        def _():
            _wait_idx(i + 1)
            _issue_data(i + 1)

        _wait_data(i)
        sl = lax.rem(i, 2)
        hb = hbuf.at[sl]
        eb = ebuf.at[sl]

        def _cmp(r, c2):
            for k in range(HH // 16):
                ds = pl.ds(k * 16, 16)
                hb[r, ds] = jnp.maximum(hb[r, ds] + eb[r, ds], 0.0)
            return c2
        lax.fori_loop(0, B, _cmp, 0)

        pltpu.sync_copy(hb, aggr_sh.at[dstb.at[lax.rem(i, 4)]], add=True)
        return carry
    lax.fori_loop(0, NBLK, _blk, 0)

    plsc.subcore_barrier()

    rows_a, rows_b = 632, N - 15 * 632

    @pl.when(s < NT - 1)
    def _():
        pltpu.sync_copy(aggr_sh.at[pl.ds(s * rows_a, rows_a)],
                        out_hbm.at[pl.ds(c * N + s * rows_a, rows_a)])

    @pl.when(s == NT - 1)
    def _():
        pltpu.sync_copy(aggr_sh.at[pl.ds(15 * rows_a, rows_b)],
                        out_hbm.at[pl.ds(c * N + 15 * rows_a, rows_b)])


@functools.cache
def _get_edge_sc():
    return pl.kernel(
        _edge_body,
        out_type=jax.ShapeDtypeStruct((NSC * N, HH), jnp.float32),
        mesh=plsc.VectorSubcoreMesh(core_axis_name="c", subcore_axis_name="s",
                                    num_cores=NSC, num_subcores=NT),
        scratch_types=[
            pltpu.VMEM((4, B), jnp.int32),
            pltpu.VMEM((4, B), jnp.int32),
            pltpu.VMEM((2, B, HH), jnp.float32),
            pltpu.VMEM((2, B, HH), jnp.float32),
            pltpu.VMEM_SHARED((AROWS, HH), jnp.float32),
            pltpu.SemaphoreType.DMA,
            pltpu.SemaphoreType.DMA,
            pltpu.SemaphoreType.DMA,
        ],
    )


_BN = 2000
_BE = 8192


def _proj_body(x_ref, w_ref, b_ref, o_ref):
    o_ref[0] = (jnp.dot(x_ref[...], w_ref[...],
                        preferred_element_type=jnp.float32) + b_ref[0])


def _node_proj(x, w, b):
    return pl.pallas_call(
        _proj_body,
        grid=(2, N // _BN),
        in_specs=[
            pl.BlockSpec((_BN, 128), lambda c, i: (i, 0)),
            pl.BlockSpec((128, HH), lambda c, i: (0, c)),
            pl.BlockSpec((1, 1, HH), lambda c, i: (c, 0, 0)),
        ],
        out_specs=pl.BlockSpec((1, _BN, HH), lambda c, i: (c, i, 0)),
        out_shape=jax.ShapeDtypeStruct((2, N, HH), jnp.float32),
        compiler_params=pltpu.CompilerParams(
            dimension_semantics=("arbitrary", "arbitrary")),
    )(x, w, b.reshape(2, 1, HH))


def _edge_proj(ea, w, b):
    return pl.pallas_call(
        _proj_body,
        grid=(2, E_PAD // _BE),
        in_specs=[
            pl.BlockSpec((_BE, 16), lambda c, i: (i, 0)),
            pl.BlockSpec((16, HH), lambda c, i: (0, c)),
            pl.BlockSpec((1, 1, HH), lambda c, i: (c, 0, 0)),
        ],
        out_specs=pl.BlockSpec((1, _BE, HH), lambda c, i: (c, i, 0)),
        out_shape=jax.ShapeDtypeStruct((2, E_PAD, HH), jnp.float32),
        compiler_params=pltpu.CompilerParams(
            dimension_semantics=("arbitrary", "arbitrary")),
    )(ea, w, b.reshape(2, 1, HH))


def _mlp_body(hs_ref, ag_ref, w1_ref, b1_ref, w2_ref, b2_ref, eps_ref,
              z2_ref, st_ref):
    i = pl.program_id(0)
    h = jnp.concatenate([hs_ref[0], hs_ref[1]], axis=1)
    a = jnp.concatenate([ag_ref[0], ag_ref[1]], axis=1)
    z = eps_ref[0, 0] * h + a
    a1 = jnp.maximum(jnp.dot(z, w1_ref[...],
                             preferred_element_type=jnp.float32)
                     + b1_ref[...], 0.0)
    z2 = jnp.dot(a1, w2_ref[...],
                 preferred_element_type=jnp.float32) + b2_ref[...]
    z2_ref[...] = z2

    @pl.when(i == 0)
    def _():
        st_ref[...] = jnp.zeros((2, H), jnp.float32)

    st_ref[0:1, :] = st_ref[0:1, :] + jnp.sum(z2, axis=0, keepdims=True)
    st_ref[1:2, :] = st_ref[1:2, :] + jnp.sum(z2 * z2, axis=0, keepdims=True)


def _mlp(h_split, aggr, w1, b1, w2, b2, eps1):
    return pl.pallas_call(
        _mlp_body,
        grid=(N // _BN,),
        in_specs=[
            pl.BlockSpec((2, _BN, HH), lambda i: (0, i, 0)),
            pl.BlockSpec((2, _BN, HH), lambda i: (0, i, 0)),
            pl.BlockSpec((H, H), lambda i: (0, 0)),
            pl.BlockSpec((1, H), lambda i: (0, 0)),
            pl.BlockSpec((H, H), lambda i: (0, 0)),
            pl.BlockSpec((1, H), lambda i: (0, 0)),
            pl.BlockSpec((1, 1), lambda i: (0, 0)),
        ],
        out_specs=[
            pl.BlockSpec((_BN, H), lambda i: (i, 0)),
            pl.BlockSpec((2, H), lambda i: (0, 0)),
        ],
        out_shape=[
            jax.ShapeDtypeStruct((N, H), jnp.float32),
            jax.ShapeDtypeStruct((2, H), jnp.float32),
        ],
        compiler_params=pltpu.CompilerParams(
            dimension_semantics=("arbitrary",)),
    )(h_split, aggr, w1, b1, w2, b2, eps1)


def _bn_body(z2_ref, st_ref, g_ref, b_ref, hs_ref, o_ref):
    inv_n = 1.0 / N
    mu = st_ref[0:1, :] * inv_n
    var = st_ref[1:2, :] * inv_n - mu * mu
    inv = lax.rsqrt(var + 1e-5)
    zn = (z2_ref[...] - mu) * inv * g_ref[...] + b_ref[...]
    r = jnp.maximum(zn, 0.0)
    o_ref[0] = r[:, :HH] + hs_ref[0]
    o_ref[1] = r[:, HH:] + hs_ref[1]


def _bn(z2, st, g, b, h_split):
    return pl.pallas_call(
        _bn_body,
        grid=(N // _BN,),
        in_specs=[
            pl.BlockSpec((_BN, H), lambda i: (i, 0)),
            pl.BlockSpec((2, H), lambda i: (0, 0)),
            pl.BlockSpec((1, H), lambda i: (0, 0)),
            pl.BlockSpec((1, H), lambda i: (0, 0)),
            pl.BlockSpec((2, _BN, HH), lambda i: (0, i, 0)),
        ],
        out_specs=pl.BlockSpec((2, _BN, HH), lambda i: (0, i, 0)),
        out_shape=jax.ShapeDtypeStruct((2, N, HH), jnp.float32),
        compiler_params=pltpu.CompilerParams(
            dimension_semantics=("arbitrary",)),
    )(z2, st, g, b, h_split)


def _pool_body(hs_ref, bt_ref, w1_ref, b1_ref, w2_ref, b2_ref,
               lo_ref, pr_ref, pd_ref, acc, cnt):
    i = pl.program_id(0)

    @pl.when(i == 0)
    def _():
        acc[...] = jnp.zeros((NG, H), jnp.float32)
        cnt[...] = jnp.zeros((NG, 1), jnp.float32)

    h = jnp.concatenate([hs_ref[0], hs_ref[1]], axis=1)
    gid = lax.broadcasted_iota(jnp.int32, (_BN, NG), 1)
    oh = (gid == bt_ref[...]).astype(jnp.float32)
    acc[...] = acc[...] + lax.dot_general(
        oh, h, (((0,), (0,)), ((), ())),
        preferred_element_type=jnp.float32)
    cnt[...] = cnt[...] + lax.dot_general(
        oh, jnp.ones((_BN, 1), jnp.float32), (((0,), (0,)), ((), ())),
        preferred_element_type=jnp.float32)

    @pl.when(i == N // _BN - 1)
    def _():
        g = acc[...] / jnp.maximum(cnt[...], 1.0)
        a1 = jnp.maximum(jnp.dot(g, w1_ref[...],
                                 preferred_element_type=jnp.float32)
                         + b1_ref[...], 0.0)
        lg = jnp.dot(a1, w2_ref[...],
                     preferred_element_type=jnp.float32) + b2_ref[...]
        pb = jax.nn.sigmoid(lg)
        lo_ref[...] = lg
        pr_ref[...] = pb
        pd_ref[...] = (pb > 0.5).astype(jnp.float32)


def _pool_cls(h_split, batch2d, w1, b1, w2, b2):
    return pl.pallas_call(
        _pool_body,
        grid=(N // _BN,),
        in_specs=[
            pl.BlockSpec((2, _BN, HH), lambda i: (0, i, 0)),
            pl.BlockSpec((_BN, 1), lambda i: (i, 0)),
            pl.BlockSpec((H, H), lambda i: (0, 0)),
            pl.BlockSpec((1, H), lambda i: (0, 0)),
            pl.BlockSpec((H, NCLS), lambda i: (0, 0)),
            pl.BlockSpec((1, NCLS), lambda i: (0, 0)),
        ],
        out_specs=[
            pl.BlockSpec((NG, NCLS), lambda i: (0, 0)),
            pl.BlockSpec((NG, NCLS), lambda i: (0, 0)),
            pl.BlockSpec((NG, NCLS), lambda i: (0, 0)),
        ],
        out_shape=[
            jax.ShapeDtypeStruct((NG, NCLS), jnp.float32),
            jax.ShapeDtypeStruct((NG, NCLS), jnp.float32),
            jax.ShapeDtypeStruct((NG, NCLS), jnp.float32),
        ],
        scratch_shapes=[
            pltpu.VMEM((NG, H), jnp.float32),
            pltpu.VMEM((NG, 1), jnp.float32),
        ],
        compiler_params=pltpu.CompilerParams(
            dimension_semantics=("arbitrary",)),
    )(h_split, batch2d, w1, b1, w2, b2)


def kernel(x, edge_index, batch, edge_attr, node_W, node_b, edge_W, edge_b,
           mlp_W1, mlp_b1, mlp_W2, mlp_b2, eps, bn_g, bn_b,
           cls_W1, cls_b1, cls_W2, cls_b2):
    ei = edge_index.astype(jnp.int32)
    src = ei[0]
    dst = ei[1]
    pad = E_PAD - E
    src3 = jnp.concatenate(
        [src, jnp.zeros((pad,), jnp.int32)]).reshape(NT, NROW, BH)
    dst3 = jnp.concatenate(
        [dst, N + (jnp.arange(pad, dtype=jnp.int32) % BH)]).reshape(NT, NROW, BH)
    eap = jnp.concatenate(
        [edge_attr, jnp.zeros((pad, edge_attr.shape[1]), jnp.float32)])
    batch2d = batch.astype(jnp.int32).reshape(N, 1)

    h_split = _node_proj(x, node_W, node_b)
    e_cat = _edge_proj(eap, edge_W, edge_b).reshape(NSC * E_PAD, HH)

    for l in range(L):
        h_cat = h_split.reshape(NSC * N, HH)
        aggr = _get_edge_sc()(h_cat, e_cat, src3, dst3).reshape(2, N, HH)
        z2, st = _mlp(h_split, aggr, mlp_W1[l], mlp_b1[l].reshape(1, H),
                      mlp_W2[l], mlp_b2[l].reshape(1, H),
                      (1.0 + eps[l]).reshape(1, 1))
        h_split = _bn(z2, st, bn_g[l].reshape(1, H), bn_b[l].reshape(1, H),
                      h_split)

    logits, probs, preds = _pool_cls(
        h_split, batch2d, cls_W1, cls_b1.reshape(1, H),
        cls_W2, cls_b2.reshape(1, NCLS))
    return (logits, probs, preds, preds)


# speedup vs baseline: 2.7665x; 1.5723x over previous
import functools

import jax
import jax.numpy as jnp
from jax import lax
from jax.experimental import pallas as pl
from jax.experimental.pallas import tpu as pltpu
import jax.experimental.pallas.tpu_sc as plsc

N = 10000
E = 320000
H = 256
HH = 128
L = 5
NG = 64
NCLS = 10

NSC = 2
NT = 16
B = 64
BH = 128
NBLK = 320
EPT = NBLK * B
NROW = EPT // BH
E_PAD = NT * EPT
AROWS = 10240


def _edge_body(h_hbm, e_hbm, src_hbm, dst_hbm, out_hbm,
               srcb, dstb, hbuf, ebuf, aggr_sh, isem, hsem, esem, ssem):
    c = lax.axis_index("c")
    s = lax.axis_index("s")
    coff = c * N
    ebase = c * E_PAD + s * EPT

    z0 = hbuf.at[0]

    def _zero(r, carry):
        for k in range(HH // 16):
            z0[r, pl.ds(k * 16, 16)] = jnp.zeros((16,), jnp.float32)
        return carry
    lax.fori_loop(0, B, _zero, 0)
    zbase = s * 640
    for k in range(640 // B):
        pltpu.sync_copy(z0, aggr_sh.at[pl.ds(zbase + k * B, B)])
    plsc.subcore_barrier()

    def _idx_src_refs(i):
        r = lax.div(i, 2)
        off = lax.rem(i, 2) * B
        return (src_hbm.at[s, r, pl.ds(off, B)], dst_hbm.at[s, r, pl.ds(off, B)])

    def _issue_idx(i):
        sl = lax.rem(i, 4)
        sh, dh = _idx_src_refs(i)
        pltpu.async_copy(sh, srcb.at[sl], isem)
        pltpu.async_copy(dh, dstb.at[sl], isem)

    def _wait_idx(i):
        sl4 = lax.rem(i, 4)
        sh, dh = _idx_src_refs(i)
        pltpu.make_async_copy(sh, srcb.at[sl4], isem).wait()
        pltpu.make_async_copy(dh, dstb.at[sl4], isem).wait()
        for slot in range(4):
            @pl.when(sl4 == slot)
            def _(slot=slot):
                sb = srcb.at[slot]
                for k in range(B // 16):
                    ds = pl.ds(k * 16, 16)
                    sb[ds] = sb[ds] + coff

    def _issue_data(i):
        pltpu.async_copy(h_hbm.at[srcb.at[lax.rem(i, 4)]],
                         hbuf.at[lax.rem(i, 3)], hsem)
        pltpu.async_copy(e_hbm.at[pl.ds(ebase + i * B, B)],
                         ebuf.at[lax.rem(i, 2)], esem)

    def _wait_data(i):
        pltpu.make_async_copy(h_hbm.at[srcb.at[lax.rem(i, 4)]],
                              hbuf.at[lax.rem(i, 3)], hsem).wait()
        pltpu.make_async_copy(e_hbm.at[pl.ds(ebase + i * B, B)],
                              ebuf.at[lax.rem(i, 2)], esem).wait()

    def _compute(i):
        m6 = lax.rem(i, 6)
        for m in range(6):
            @pl.when(m6 == m)
            def _(m=m):
                hb = hbuf.at[m % 3]
                eb = ebuf.at[m % 2]

                def _row(r, carry):
                    hv = [hb[r, pl.ds(k * 16, 16)] for k in range(HH // 16)]
                    ev = [eb[r, pl.ds(k * 16, 16)] for k in range(HH // 16)]
                    for k in range(HH // 16):
                        hb[r, pl.ds(k * 16, 16)] = jnp.maximum(
                            hv[k] + ev[k], 0.0)
                    return carry
                lax.fori_loop(0, B, _row, 0, unroll=2)

    def _issue_scatter(i):
        pltpu.async_copy(hbuf.at[lax.rem(i, 3)],
                         aggr_sh.at[dstb.at[lax.rem(i, 4)]], ssem, add=True)

    def _wait_scatter(i):
        pltpu.make_async_copy(hbuf.at[lax.rem(i, 3)],
                              aggr_sh.at[dstb.at[lax.rem(i, 4)]], ssem).wait()

    _issue_idx(0)
    _issue_idx(1)
    _wait_idx(0)
    _issue_data(0)

    def _blk(i, carry):
        @pl.when(i + 2 < NBLK)
        def _():
            _issue_idx(i + 2)

        @pl.when(i + 1 < NBLK)
        def _():
            _wait_idx(i + 1)

        @pl.when(i >= 2)
        def _():
            _wait_scatter(i - 2)

        @pl.when(i + 1 < NBLK)
        def _():
            _issue_data(i + 1)

        _wait_data(i)
        _compute(i)
        _issue_scatter(i)
        return carry
    lax.fori_loop(0, NBLK, _blk, 0)

    _wait_scatter(NBLK - 2)
    _wait_scatter(NBLK - 1)
    plsc.subcore_barrier()

    rows_a, rows_b = 632, N - 15 * 632

    @pl.when(s < NT - 1)
    def _():
        pltpu.sync_copy(aggr_sh.at[pl.ds(s * rows_a, rows_a)],
                        out_hbm.at[pl.ds(c * N + s * rows_a, rows_a)])

    @pl.when(s == NT - 1)
    def _():
        pltpu.sync_copy(aggr_sh.at[pl.ds(15 * rows_a, rows_b)],
                        out_hbm.at[pl.ds(c * N + 15 * rows_a, rows_b)])


@functools.cache
def _get_edge_sc():
    return pl.kernel(
        _edge_body,
        out_type=jax.ShapeDtypeStruct((NSC * N, HH), jnp.float32),
        mesh=plsc.VectorSubcoreMesh(core_axis_name="c", subcore_axis_name="s",
                                    num_cores=NSC, num_subcores=NT),
        scratch_types=[
            pltpu.VMEM((4, B), jnp.int32),
            pltpu.VMEM((4, B), jnp.int32),
            pltpu.VMEM((3, B, HH), jnp.float32),
            pltpu.VMEM((2, B, HH), jnp.float32),
            pltpu.VMEM_SHARED((AROWS, HH), jnp.float32),
            pltpu.SemaphoreType.DMA,
            pltpu.SemaphoreType.DMA,
            pltpu.SemaphoreType.DMA,
            pltpu.SemaphoreType.DMA,
        ],
    )


_BN = 2000
_BE = 8192


def _proj_body(x_ref, w_ref, b_ref, o_ref):
    o_ref[0] = (jnp.dot(x_ref[...], w_ref[...],
                        preferred_element_type=jnp.float32) + b_ref[0])


def _node_proj(x, w, b):
    return pl.pallas_call(
        _proj_body,
        grid=(2, N // _BN),
        in_specs=[
            pl.BlockSpec((_BN, 128), lambda c, i: (i, 0)),
            pl.BlockSpec((128, HH), lambda c, i: (0, c)),
            pl.BlockSpec((1, 1, HH), lambda c, i: (c, 0, 0)),
        ],
        out_specs=pl.BlockSpec((1, _BN, HH), lambda c, i: (c, i, 0)),
        out_shape=jax.ShapeDtypeStruct((2, N, HH), jnp.float32),
        compiler_params=pltpu.CompilerParams(
            dimension_semantics=("arbitrary", "arbitrary")),
    )(x, w, b.reshape(2, 1, HH))


def _edge_proj(ea, w, b):
    return pl.pallas_call(
        _proj_body,
        grid=(2, E_PAD // _BE),
        in_specs=[
            pl.BlockSpec((_BE, 16), lambda c, i: (i, 0)),
            pl.BlockSpec((16, HH), lambda c, i: (0, c)),
            pl.BlockSpec((1, 1, HH), lambda c, i: (c, 0, 0)),
        ],
        out_specs=pl.BlockSpec((1, _BE, HH), lambda c, i: (c, i, 0)),
        out_shape=jax.ShapeDtypeStruct((2, E_PAD, HH), jnp.float32),
        compiler_params=pltpu.CompilerParams(
            dimension_semantics=("arbitrary", "arbitrary")),
    )(ea, w, b.reshape(2, 1, HH))


def _mlp_body(hs_ref, ag_ref, w1_ref, b1_ref, w2_ref, b2_ref, eps_ref,
              z2_ref, st_ref):
    i = pl.program_id(0)
    h = jnp.concatenate([hs_ref[0], hs_ref[1]], axis=1)
    a = jnp.concatenate([ag_ref[0], ag_ref[1]], axis=1)
    z = eps_ref[0, 0] * h + a
    a1 = jnp.maximum(jnp.dot(z, w1_ref[...],
                             preferred_element_type=jnp.float32)
                     + b1_ref[...], 0.0)
    z2 = jnp.dot(a1, w2_ref[...],
                 preferred_element_type=jnp.float32) + b2_ref[...]
    z2_ref[...] = z2

    @pl.when(i == 0)
    def _():
        st_ref[...] = jnp.zeros((2, H), jnp.float32)

    st_ref[0:1, :] = st_ref[0:1, :] + jnp.sum(z2, axis=0, keepdims=True)
    st_ref[1:2, :] = st_ref[1:2, :] + jnp.sum(z2 * z2, axis=0, keepdims=True)


def _mlp(h_split, aggr, w1, b1, w2, b2, eps1):
    return pl.pallas_call(
        _mlp_body,
        grid=(N // _BN,),
        in_specs=[
            pl.BlockSpec((2, _BN, HH), lambda i: (0, i, 0)),
            pl.BlockSpec((2, _BN, HH), lambda i: (0, i, 0)),
            pl.BlockSpec((H, H), lambda i: (0, 0)),
            pl.BlockSpec((1, H), lambda i: (0, 0)),
            pl.BlockSpec((H, H), lambda i: (0, 0)),
            pl.BlockSpec((1, H), lambda i: (0, 0)),
            pl.BlockSpec((1, 1), lambda i: (0, 0)),
        ],
        out_specs=[
            pl.BlockSpec((_BN, H), lambda i: (i, 0)),
            pl.BlockSpec((2, H), lambda i: (0, 0)),
        ],
        out_shape=[
            jax.ShapeDtypeStruct((N, H), jnp.float32),
            jax.ShapeDtypeStruct((2, H), jnp.float32),
        ],
        compiler_params=pltpu.CompilerParams(
            dimension_semantics=("arbitrary",)),
    )(h_split, aggr, w1, b1, w2, b2, eps1)


def _bn_body(z2_ref, st_ref, g_ref, b_ref, hs_ref, o_ref):
    inv_n = 1.0 / N
    mu = st_ref[0:1, :] * inv_n
    var = st_ref[1:2, :] * inv_n - mu * mu
    inv = lax.rsqrt(var + 1e-5)
    zn = (z2_ref[...] - mu) * inv * g_ref[...] + b_ref[...]
    r = jnp.maximum(zn, 0.0)
    o_ref[0] = r[:, :HH] + hs_ref[0]
    o_ref[1] = r[:, HH:] + hs_ref[1]


def _bn(z2, st, g, b, h_split):
    return pl.pallas_call(
        _bn_body,
        grid=(N // _BN,),
        in_specs=[
            pl.BlockSpec((_BN, H), lambda i: (i, 0)),
            pl.BlockSpec((2, H), lambda i: (0, 0)),
            pl.BlockSpec((1, H), lambda i: (0, 0)),
            pl.BlockSpec((1, H), lambda i: (0, 0)),
            pl.BlockSpec((2, _BN, HH), lambda i: (0, i, 0)),
        ],
        out_specs=pl.BlockSpec((2, _BN, HH), lambda i: (0, i, 0)),
        out_shape=jax.ShapeDtypeStruct((2, N, HH), jnp.float32),
        compiler_params=pltpu.CompilerParams(
            dimension_semantics=("arbitrary",)),
    )(z2, st, g, b, h_split)


def _pool_body(hs_ref, bt_ref, w1_ref, b1_ref, w2_ref, b2_ref,
               lo_ref, pr_ref, pd_ref, acc, cnt):
    i = pl.program_id(0)

    @pl.when(i == 0)
    def _():
        acc[...] = jnp.zeros((NG, H), jnp.float32)
        cnt[...] = jnp.zeros((NG, 1), jnp.float32)

    h = jnp.concatenate([hs_ref[0], hs_ref[1]], axis=1)
    gid = lax.broadcasted_iota(jnp.int32, (_BN, NG), 1)
    oh = (gid == bt_ref[...]).astype(jnp.float32)
    acc[...] = acc[...] + lax.dot_general(
        oh, h, (((0,), (0,)), ((), ())),
        preferred_element_type=jnp.float32)
    cnt[...] = cnt[...] + lax.dot_general(
        oh, jnp.ones((_BN, 1), jnp.float32), (((0,), (0,)), ((), ())),
        preferred_element_type=jnp.float32)

    @pl.when(i == N // _BN - 1)
    def _():
        g = acc[...] / jnp.maximum(cnt[...], 1.0)
        a1 = jnp.maximum(jnp.dot(g, w1_ref[...],
                                 preferred_element_type=jnp.float32)
                         + b1_ref[...], 0.0)
        lg = jnp.dot(a1, w2_ref[...],
                     preferred_element_type=jnp.float32) + b2_ref[...]
        pb = jax.nn.sigmoid(lg)
        lo_ref[...] = lg
        pr_ref[...] = pb
        pd_ref[...] = (pb > 0.5).astype(jnp.float32)


def _pool_cls(h_split, batch2d, w1, b1, w2, b2):
    return pl.pallas_call(
        _pool_body,
        grid=(N // _BN,),
        in_specs=[
            pl.BlockSpec((2, _BN, HH), lambda i: (0, i, 0)),
            pl.BlockSpec((_BN, 1), lambda i: (i, 0)),
            pl.BlockSpec((H, H), lambda i: (0, 0)),
            pl.BlockSpec((1, H), lambda i: (0, 0)),
            pl.BlockSpec((H, NCLS), lambda i: (0, 0)),
            pl.BlockSpec((1, NCLS), lambda i: (0, 0)),
        ],
        out_specs=[
            pl.BlockSpec((NG, NCLS), lambda i: (0, 0)),
            pl.BlockSpec((NG, NCLS), lambda i: (0, 0)),
            pl.BlockSpec((NG, NCLS), lambda i: (0, 0)),
        ],
        out_shape=[
            jax.ShapeDtypeStruct((NG, NCLS), jnp.float32),
            jax.ShapeDtypeStruct((NG, NCLS), jnp.float32),
            jax.ShapeDtypeStruct((NG, NCLS), jnp.float32),
        ],
        scratch_shapes=[
            pltpu.VMEM((NG, H), jnp.float32),
            pltpu.VMEM((NG, 1), jnp.float32),
        ],
        compiler_params=pltpu.CompilerParams(
            dimension_semantics=("arbitrary",)),
    )(h_split, batch2d, w1, b1, w2, b2)


def kernel(x, edge_index, batch, edge_attr, node_W, node_b, edge_W, edge_b,
           mlp_W1, mlp_b1, mlp_W2, mlp_b2, eps, bn_g, bn_b,
           cls_W1, cls_b1, cls_W2, cls_b2):
    ei = edge_index.astype(jnp.int32)
    src = ei[0]
    dst = ei[1]
    pad = E_PAD - E
    src3 = jnp.concatenate(
        [src, jnp.zeros((pad,), jnp.int32)]).reshape(NT, NROW, BH)
    dst3 = jnp.concatenate(
        [dst, N + (jnp.arange(pad, dtype=jnp.int32) % BH)]).reshape(NT, NROW, BH)
    eap = jnp.concatenate(
        [edge_attr, jnp.zeros((pad, edge_attr.shape[1]), jnp.float32)])
    batch2d = batch.astype(jnp.int32).reshape(N, 1)

    h_split = _node_proj(x, node_W, node_b)
    e_cat = _edge_proj(eap, edge_W, edge_b).reshape(NSC * E_PAD, HH)

    for l in range(L):
        h_cat = h_split.reshape(NSC * N, HH)
        aggr = _get_edge_sc()(h_cat, e_cat, src3, dst3).reshape(2, N, HH)
        z2, st = _mlp(h_split, aggr, mlp_W1[l], mlp_b1[l].reshape(1, H),
                      mlp_W2[l], mlp_b2[l].reshape(1, H),
                      (1.0 + eps[l]).reshape(1, 1))
        h_split = _bn(z2, st, bn_g[l].reshape(1, H), bn_b[l].reshape(1, H),
                      h_split)

    logits, probs, preds = _pool_cls(
        h_split, batch2d, cls_W1, cls_b1.reshape(1, H),
        cls_W2, cls_b2.reshape(1, NCLS))
    return (logits, probs, preds, preds)
